# baseline (speedup 1.0000x reference)
import functools

import jax
import jax.numpy as jnp
from jax import lax
from jax.experimental import pallas as pl
from jax.experimental.pallas import tpu as pltpu
from jax.experimental.pallas import tpu_sc as plsc

H = 128
DEPTH = 5
NC = 2
NS = 16
NW = NC * NS
F32 = jnp.float32


def _mm_body(x_ref, w_ref, o_ref):
    o_ref[...] = jnp.dot(x_ref[...], w_ref[...], preferred_element_type=F32)


def _matmul(x, w, blk):
    m, k = x.shape
    n = w.shape[1]
    return pl.pallas_call(
        _mm_body,
        grid=(m // blk,),
        in_specs=[
            pl.BlockSpec((blk, k), lambda i: (i, 0)),
            pl.BlockSpec((k, n), lambda i: (0, 0)),
        ],
        out_specs=pl.BlockSpec((blk, n), lambda i: (i, 0)),
        out_shape=jax.ShapeDtypeStruct((m, n), F32),
    )(x, w)


def _mm_relu_body(x_ref, w_ref, inp_ref, msg_ref):
    acc = jnp.dot(x_ref[...], w_ref[...], preferred_element_type=F32)
    inp_ref[...] = acc
    msg_ref[...] = jnp.maximum(acc, 0.0)


def _input_matmul(f_bonds, W_i, blk):
    m, k = f_bonds.shape
    n = W_i.shape[1]
    shp = jax.ShapeDtypeStruct((m, n), F32)
    return pl.pallas_call(
        _mm_relu_body,
        grid=(m // blk,),
        in_specs=[
            pl.BlockSpec((blk, k), lambda i: (i, 0)),
            pl.BlockSpec((k, n), lambda i: (0, 0)),
        ],
        out_specs=[
            pl.BlockSpec((blk, n), lambda i: (i, 0)),
            pl.BlockSpec((blk, n), lambda i: (i, 0)),
        ],
        out_shape=[shp, shp],
    )(f_bonds, W_i)


def _fuse_body(inp_ref, pre_ref, w_ref, msg_ref, h_ref):
    m = jnp.maximum(inp_ref[...] + pre_ref[...], 0.0)
    msg_ref[...] = m
    h_ref[...] = jnp.dot(m, w_ref[...], preferred_element_type=F32)


def _fuse(inp, pre, w, blk):
    m, n = inp.shape
    shp = jax.ShapeDtypeStruct((m, n), F32)
    return pl.pallas_call(
        _fuse_body,
        grid=(m // blk,),
        in_specs=[
            pl.BlockSpec((blk, n), lambda i: (i, 0)),
            pl.BlockSpec((blk, n), lambda i: (i, 0)),
            pl.BlockSpec((n, n), lambda i: (0, 0)),
        ],
        out_specs=[
            pl.BlockSpec((blk, n), lambda i: (i, 0)),
            pl.BlockSpec((blk, n), lambda i: (i, 0)),
        ],
        out_shape=[shp, shp],
    )(inp, pre, w)


def _fuse_last_body(inp_ref, pre_ref, msg_ref):
    msg_ref[...] = jnp.maximum(inp_ref[...] + pre_ref[...], 0.0)


def _fuse_last(inp, pre, blk):
    m, n = inp.shape
    return pl.pallas_call(
        _fuse_last_body,
        grid=(m // blk,),
        in_specs=[
            pl.BlockSpec((blk, n), lambda i: (i, 0)),
            pl.BlockSpec((blk, n), lambda i: (i, 0)),
        ],
        out_specs=pl.BlockSpec((blk, n), lambda i: (i, 0)),
        out_shape=jax.ShapeDtypeStruct((m, n), F32),
    )(inp, pre)


MOLP = 512


def _readout_body(fa_ref, am_ref, ids_ref, wo1_ref, wo2_ref, bo_ref,
                  out_ref, cnt_ref):
    i = pl.program_id(0)
    hidden = jnp.maximum(
        jnp.dot(fa_ref[...], wo1_ref[...], preferred_element_type=F32)
        + jnp.dot(am_ref[...], wo2_ref[...], preferred_element_type=F32)
        + bo_ref[...],
        0.0,
    )
    ids = ids_ref[0, 0, :]
    onehot = (lax.broadcasted_iota(jnp.int32, (MOLP, ids.shape[0]), 0)
              == ids[None, :]).astype(F32)
    part = jnp.dot(onehot, hidden, preferred_element_type=F32)
    cpart = jnp.sum(onehot, axis=1, keepdims=True)

    @pl.when(i == 0)
    def _():
        out_ref[...] = jnp.zeros_like(out_ref)
        cnt_ref[...] = jnp.zeros_like(cnt_ref)

    out_ref[...] += part
    cnt_ref[...] += jnp.broadcast_to(cpart, cnt_ref.shape)

    @pl.when(i == pl.num_programs(0) - 1)
    def _():
        out_ref[...] = out_ref[...] / jnp.maximum(cnt_ref[...], 1.0)


def _readout(f_atoms, a_msg, ids3, Wo1, Wo2, b_o, blk):
    na, fa = f_atoms.shape
    grid = na // blk
    return pl.pallas_call(
        _readout_body,
        grid=(grid,),
        in_specs=[
            pl.BlockSpec((blk, fa), lambda i: (i, 0)),
            pl.BlockSpec((blk, H), lambda i: (i, 0)),
            pl.BlockSpec((1, 1, blk), lambda i: (i, 0, 0)),
            pl.BlockSpec((fa, H), lambda i: (0, 0)),
            pl.BlockSpec((H, H), lambda i: (0, 0)),
            pl.BlockSpec((1, H), lambda i: (0, 0)),
        ],
        out_specs=pl.BlockSpec((MOLP, H), lambda i: (0, 0)),
        out_shape=jax.ShapeDtypeStruct((MOLP, H), F32),
        scratch_shapes=[pltpu.VMEM((MOLP, H), F32)],
    )(f_atoms, a_msg, ids3, Wo1, Wo2, b_o)


_MESH = plsc.VectorSubcoreMesh(core_axis_name="c", subcore_axis_name="s")

NAP = 10240
APS = NAP // NC
APT = APS // NS
GROWS = 128
GCHUNKS = APT * 32 // GROWS
GL = 16


def _sum_chunk(rows, k, acc_v):
    for a in range(4):
        for g in range(H // GL):
            sl = pl.ds(g * GL, GL)
            vals = [rows[a * 32 + r, sl] for r in range(32)]
            while len(vals) > 1:
                vals = [vals[i] + vals[i + 1] for i in range(0, len(vals), 2)]
            acc_v[k * 4 + a, sl] = vals[0]


def _gather_sum_sc(msg_hbm, a2b_hbm, out_hbm,
                   idx_v, rows_a, rows_b, acc_v, sem_a, sem_b):
    c = lax.axis_index("c")
    s = lax.axis_index("s")
    pltpu.sync_copy(
        a2b_hbm.at[pl.ds(c * (APS * 32 // GROWS) + s * GCHUNKS, GCHUNKS)],
        idx_v)
    pltpu.async_copy(msg_hbm.at[idx_v.at[0]], rows_a, sem_a)
    pltpu.async_copy(msg_hbm.at[idx_v.at[1]], rows_b, sem_b)

    def body(j, _):
        k0 = 2 * j
        pltpu.make_async_copy(msg_hbm.at[idx_v.at[k0]], rows_a, sem_a).wait()
        _sum_chunk(rows_a, k0, acc_v)

        @pl.when(k0 + 2 < GCHUNKS)
        def _():
            pltpu.async_copy(msg_hbm.at[idx_v.at[k0 + 2]], rows_a, sem_a)

        pltpu.make_async_copy(msg_hbm.at[idx_v.at[k0 + 1]], rows_b,
                              sem_b).wait()
        _sum_chunk(rows_b, k0 + 1, acc_v)

        @pl.when(k0 + 3 < GCHUNKS)
        def _():
            pltpu.async_copy(msg_hbm.at[idx_v.at[k0 + 3]], rows_b, sem_b)

        return 0

    lax.fori_loop(0, GCHUNKS // 2, body, 0)
    pltpu.sync_copy(acc_v, out_hbm.at[pl.ds(c * APS + s * APT, APT)])


@functools.partial(
    pl.kernel,
    out_type=jax.ShapeDtypeStruct((NAP, H), F32),
    mesh=_MESH,
    scratch_types=[
        pltpu.VMEM((GCHUNKS, GROWS), jnp.int32),
        pltpu.VMEM((GROWS, H), F32),
        pltpu.VMEM((GROWS, H), F32),
        pltpu.VMEM((APT, H), F32),
        pltpu.SemaphoreType.DMA,
        pltpu.SemaphoreType.DMA,
    ],
)
def _gather_sum(msg_hbm, a2b_hbm, out_hbm,
                idx_v, rows_a, rows_b, acc_v, sem_a, sem_b):
    _gather_sum_sc(msg_hbm, a2b_hbm, out_hbm,
                   idx_v, rows_a, rows_b, acc_v, sem_a, sem_b)


CCH = 128
NCK = 2528
CPW = NCK // NW


def _diff_chunk(am_v, h_v, out_v):
    def blk(b, _):
        r0 = b * 8
        for r in range(8):
            for g in range(H // GL):
                sl = pl.ds(g * GL, GL)
                out_v[r0 + r, sl] = am_v[r0 + r, sl] - h_v[r0 + r, sl]
        return 0

    lax.fori_loop(0, CCH // 8, blk, 0)


def _combine_sc(am_hbm, h_hbm, b2a_hbm, brev_hbm, out_hbm,
                idxa_bufs, idxr_bufs, am_bufs, h_bufs, out_bufs,
                sems_a, sems_h, sems_o):
    c = lax.axis_index("c")
    s = lax.axis_index("s")
    w = s * NC + c

    def start(slot, i):
        pltpu.sync_copy(b2a_hbm.at[w, i], idxa_bufs[slot])
        pltpu.sync_copy(brev_hbm.at[w, i], idxr_bufs[slot])
        pltpu.async_copy(am_hbm.at[idxa_bufs[slot]], am_bufs[slot],
                         sems_a[slot])
        pltpu.async_copy(h_hbm.at[idxr_bufs[slot]], h_bufs[slot],
                         sems_h[slot])

    def wait_gather(slot, i):
        pltpu.make_async_copy(am_hbm.at[idxa_bufs[slot]], am_bufs[slot],
                              sems_a[slot]).wait()
        pltpu.make_async_copy(h_hbm.at[idxr_bufs[slot]], h_bufs[slot],
                              sems_h[slot]).wait()

    def wait_store(slot):
        pltpu.make_async_copy(out_bufs[slot], out_hbm.at[pl.ds(0, CCH)],
                              sems_o[slot]).wait()

    start(0, 0)
    start(1, 1)

    def body(j, _):
        for slot in range(2):
            i = 2 * j + slot
            ck = i * NW + w
            wait_gather(slot, i)

            @pl.when(j > 0)
            def _():
                wait_store(slot)

            _diff_chunk(am_bufs[slot], h_bufs[slot], out_bufs[slot])

            @pl.when(i + 2 < CPW)
            def _():
                start(slot, i + 2)

            pltpu.async_copy(out_bufs[slot],
                             out_hbm.at[pl.ds(ck * CCH, CCH)],
                             sems_o[slot])
        return 0

    lax.fori_loop(0, (CPW - 1) // 2, body, 0)
    i = CPW - 1
    wait_gather(0, i)
    wait_store(0)
    _diff_chunk(am_bufs[0], h_bufs[0], out_bufs[0])
    pltpu.async_copy(out_bufs[0],
                     out_hbm.at[pl.ds((i * NW + w) * CCH, CCH)],
                     sems_o[0])
    for slot in range(2):
        wait_store(slot)


@functools.partial(
    pl.kernel,
    out_type=jax.ShapeDtypeStruct((NCK * CCH, H), F32),
    mesh=_MESH,
    scratch_types=[
        pltpu.VMEM((CCH,), jnp.int32),
        pltpu.VMEM((CCH,), jnp.int32),
        pltpu.VMEM((CCH,), jnp.int32),
        pltpu.VMEM((CCH,), jnp.int32),
        pltpu.VMEM((CCH, H), F32),
        pltpu.VMEM((CCH, H), F32),
        pltpu.VMEM((CCH, H), F32),
        pltpu.VMEM((CCH, H), F32),
        pltpu.VMEM((CCH, H), F32),
        pltpu.VMEM((CCH, H), F32),
        pltpu.SemaphoreType.DMA,
        pltpu.SemaphoreType.DMA,
        pltpu.SemaphoreType.DMA,
        pltpu.SemaphoreType.DMA,
        pltpu.SemaphoreType.DMA,
        pltpu.SemaphoreType.DMA,
    ],
)
def _combine(am_hbm, h_hbm, b2a_hbm, brev_hbm, out_hbm,
             ia0, ia1, ir0, ir1, am0, am1, h0, h1, o0, o1,
             sa0, sa1, sh0, sh1, so0, so1):
    _combine_sc(am_hbm, h_hbm, b2a_hbm, brev_hbm, out_hbm,
                (ia0, ia1), (ir0, ir1), (am0, am1), (h0, h1), (o0, o1),
                (sa0, sa1), (sh0, sh1), (so0, so1))


def kernel(f_atoms, f_bonds, a2b, b2a, b2revb, mol_ids, W_i, W_h, W_o, b_o):
    na, fa_dim = f_atoms.shape
    nb = f_bonds.shape[0]
    maxnb = a2b.shape[1]

    a2b_p = jnp.pad(a2b.astype(jnp.int32), ((0, NAP - na), (0, 0)))
    a2b_rs = a2b_p.reshape(NAP * maxnb // GROWS, GROWS)

    def ck_arrange(x):
        xp = jnp.pad(x.astype(jnp.int32), (0, NCK * CCH - nb))
        return xp.reshape(CPW, NW, CCH).transpose(1, 0, 2)

    b2a_rs = ck_arrange(b2a)
    brev_rs = ck_arrange(b2revb)
    ids3 = mol_ids.astype(jnp.int32).reshape(5, 1, na // 5)
    Wo1 = W_o[:fa_dim]
    Wo2 = W_o[fa_dim:]
    bo2 = b_o.reshape(1, H)

    inp, msg = _input_matmul(f_bonds, W_i, 1600)

    for t in range(DEPTH - 1):
        ga = _gather_sum(msg, a2b_rs)
        h = _matmul(msg, W_h, 1600)
        am_h = _matmul(ga, W_h, 2048)
        pre = _combine(am_h, h, b2a_rs, brev_rs)[:nb]
        msg = _fuse_last(inp, pre, 1600)

    ga = _gather_sum(msg, a2b_rs)
    out = _readout(f_atoms, ga[:na], ids3, Wo1, Wo2, bo2, 2000)
    n_mols = 500
    return out[:n_mols]


# speedup vs baseline: 1.1470x; 1.1470x over previous
import functools

import jax
import jax.numpy as jnp
from jax import lax
from jax.experimental import pallas as pl
from jax.experimental.pallas import tpu as pltpu
from jax.experimental.pallas import tpu_sc as plsc

H = 128
DEPTH = 5
NC = 2
NS = 16
NW = NC * NS


def _mm_body(x_ref, w_ref, o_ref):
    o_ref[...] = jnp.dot(x_ref[...], w_ref[...],
                         preferred_element_type=jnp.float32)


def _matmul(x, w, blk):
    m, k = x.shape
    n = w.shape[1]
    return pl.pallas_call(
        _mm_body,
        grid=(m // blk,),
        in_specs=[
            pl.BlockSpec((blk, k), lambda i: (i, 0)),
            pl.BlockSpec((k, n), lambda i: (0, 0)),
        ],
        out_specs=pl.BlockSpec((blk, n), lambda i: (i, 0)),
        out_shape=jax.ShapeDtypeStruct((m, n), jnp.float32),
    )(x, w)


def _mm_relu_body(x_ref, w_ref, inp_ref, msg_ref):
    acc = jnp.dot(x_ref[...], w_ref[...], preferred_element_type=jnp.float32)
    inp_ref[...] = acc
    msg_ref[...] = jnp.maximum(acc, 0.0)


def _input_matmul(f_bonds, W_i, blk):
    m, k = f_bonds.shape
    n = W_i.shape[1]
    shp = jax.ShapeDtypeStruct((m, n), jnp.float32)
    return pl.pallas_call(
        _mm_relu_body,
        grid=(m // blk,),
        in_specs=[
            pl.BlockSpec((blk, k), lambda i: (i, 0)),
            pl.BlockSpec((k, n), lambda i: (0, 0)),
        ],
        out_specs=[
            pl.BlockSpec((blk, n), lambda i: (i, 0)),
            pl.BlockSpec((blk, n), lambda i: (i, 0)),
        ],
        out_shape=[shp, shp],
    )(f_bonds, W_i)


MOLP = 512


def _readout_body(fa_ref, am_ref, ids_ref, wo1_ref, wo2_ref, bo_ref,
                  out_ref, cnt_ref):
    i = pl.program_id(0)
    hidden = jnp.maximum(
        jnp.dot(fa_ref[...], wo1_ref[...], preferred_element_type=jnp.float32)
        + jnp.dot(am_ref[...], wo2_ref[...],
                  preferred_element_type=jnp.float32)
        + bo_ref[...],
        0.0,
    )
    ids = ids_ref[0, 0, :]
    onehot = (lax.broadcasted_iota(jnp.int32, (MOLP, ids.shape[0]), 0)
              == ids[None, :]).astype(jnp.float32)
    part = jnp.dot(onehot, hidden, preferred_element_type=jnp.float32)
    cpart = jnp.sum(onehot, axis=1, keepdims=True)

    @pl.when(i == 0)
    def _():
        out_ref[...] = jnp.zeros_like(out_ref)
        cnt_ref[...] = jnp.zeros_like(cnt_ref)

    out_ref[...] += part
    cnt_ref[...] += jnp.broadcast_to(cpart, cnt_ref.shape)

    @pl.when(i == pl.num_programs(0) - 1)
    def _():
        out_ref[...] = out_ref[...] / jnp.maximum(cnt_ref[...], 1.0)


def _readout(f_atoms, a_msg, ids3, Wo1, Wo2, b_o, blk):
    na, fa = f_atoms.shape
    grid = na // blk
    return pl.pallas_call(
        _readout_body,
        grid=(grid,),
        in_specs=[
            pl.BlockSpec((blk, fa), lambda i: (i, 0)),
            pl.BlockSpec((blk, H), lambda i: (i, 0)),
            pl.BlockSpec((1, 1, blk), lambda i: (i, 0, 0)),
            pl.BlockSpec((fa, H), lambda i: (0, 0)),
            pl.BlockSpec((H, H), lambda i: (0, 0)),
            pl.BlockSpec((1, H), lambda i: (0, 0)),
        ],
        out_specs=pl.BlockSpec((MOLP, H), lambda i: (0, 0)),
        out_shape=jax.ShapeDtypeStruct((MOLP, H), jnp.float32),
        scratch_shapes=[pltpu.VMEM((MOLP, H), jnp.float32)],
    )(f_atoms, a_msg, ids3, Wo1, Wo2, b_o)


_MESH = plsc.VectorSubcoreMesh(core_axis_name="c", subcore_axis_name="s")

NAP = 10240
APS = NAP // NC
APT = APS // NS
GROWS = 128
GCHUNKS = APT * 32 // GROWS


def _gather_sum_sc(msg_hbm, a2b_hbm, aidx_hbm, zeros_hbm, out_hbm,
                   idx_v, aidx_v, rows_a, rows_b, acc_sh, sem_a, sem_b):
    c = lax.axis_index("c")
    s = lax.axis_index("s")
    pltpu.sync_copy(a2b_hbm.at[pl.ds(c * (APS * 32 // GROWS) + s * GCHUNKS,
                                     GCHUNKS)], idx_v)
    pltpu.sync_copy(aidx_hbm.at[pl.ds(s * GCHUNKS, GCHUNKS)], aidx_v)
    pltpu.sync_copy(zeros_hbm, acc_sh.at[pl.ds(s * APT, APT)])

    def body(j, _):
        k0 = 2 * j
        d_a = pltpu.async_copy(msg_hbm.at[idx_v.at[k0]], rows_a, sem_a)
        d_b = pltpu.async_copy(msg_hbm.at[idx_v.at[k0 + 1]], rows_b, sem_b)
        d_a.wait()
        pltpu.sync_copy(rows_a, acc_sh.at[aidx_v.at[k0]], add=True)
        d_b.wait()
        pltpu.sync_copy(rows_b, acc_sh.at[aidx_v.at[k0 + 1]], add=True)
        return 0

    lax.fori_loop(0, GCHUNKS // 2, body, 0)
    pltpu.sync_copy(acc_sh.at[pl.ds(s * APT, APT)],
                    out_hbm.at[pl.ds(c * APS + s * APT, APT)])


@functools.partial(
    pl.kernel,
    out_type=jax.ShapeDtypeStruct((NAP, H), jnp.float32),
    mesh=_MESH,
    scratch_types=[
        pltpu.VMEM((GCHUNKS, GROWS), jnp.int32),
        pltpu.VMEM((GCHUNKS, GROWS), jnp.int32),
        pltpu.VMEM((GROWS, H), jnp.float32),
        pltpu.VMEM((GROWS, H), jnp.float32),
        pltpu.VMEM_SHARED((APS, H), jnp.float32),
        pltpu.SemaphoreType.DMA,
        pltpu.SemaphoreType.DMA,
    ],
)
def _gather_sum(msg_hbm, a2b_hbm, aidx_hbm, zeros_hbm, out_hbm,
                idx_v, aidx_v, rows_a, rows_b, acc_sh, sem_a, sem_b):
    _gather_sum_sc(msg_hbm, a2b_hbm, aidx_hbm, zeros_hbm, out_hbm,
                   idx_v, aidx_v, rows_a, rows_b, acc_sh, sem_a, sem_b)


CB = 80


def _combine_sc(nb, inp_hbm, am_hbm, h_hbm, b2a_hbm, brev_hbm, out_hbm,
                idxa_v, idxr_v, inp_v, am_v, h_v, out_v,
                sem_i, sem_a, sem_h):
    c = lax.axis_index("c")
    s = lax.axis_index("s")
    w = s * NC + c
    bpw = nb // NW
    nchunks = bpw // CB
    pltpu.sync_copy(b2a_hbm.at[w], idxa_v)
    pltpu.sync_copy(brev_hbm.at[w], idxr_v)

    def body(j, _):
        base = w * bpw + j * CB
        d_i = pltpu.async_copy(inp_hbm.at[pl.ds(base, CB)], inp_v, sem_i)
        d_a = pltpu.async_copy(am_hbm.at[idxa_v.at[j]], am_v, sem_a)
        d_h = pltpu.async_copy(h_hbm.at[idxr_v.at[j]], h_v, sem_h)
        d_i.wait()
        d_a.wait()
        d_h.wait()

        def row(r, _):
            for g in range(H // 16):
                sl = pl.ds(g * 16, 16)
                out_v[r, sl] = jnp.maximum(
                    inp_v[r, sl] + am_v[r, sl] - h_v[r, sl], 0.0)
            return 0

        lax.fori_loop(0, CB, row, 0)
        pltpu.sync_copy(out_v, out_hbm.at[pl.ds(base, CB)])
        return 0

    lax.fori_loop(0, nchunks, body, 0)


def _make_combine(nb):
    nchunks = nb // NW // CB

    @functools.partial(
        pl.kernel,
        out_type=jax.ShapeDtypeStruct((nb, H), jnp.float32),
        mesh=_MESH,
        scratch_types=[
            pltpu.VMEM((nchunks, CB), jnp.int32),
            pltpu.VMEM((nchunks, CB), jnp.int32),
            pltpu.VMEM((CB, H), jnp.float32),
            pltpu.VMEM((CB, H), jnp.float32),
            pltpu.VMEM((CB, H), jnp.float32),
            pltpu.VMEM((CB, H), jnp.float32),
            pltpu.SemaphoreType.DMA,
            pltpu.SemaphoreType.DMA,
            pltpu.SemaphoreType.DMA,
        ],
    )
    def _combine(inp_hbm, am_hbm, h_hbm, b2a_hbm, brev_hbm, out_hbm,
                 idxa_v, idxr_v, inp_v, am_v, h_v, out_v,
                 sem_i, sem_a, sem_h):
        _combine_sc(nb, inp_hbm, am_hbm, h_hbm, b2a_hbm, brev_hbm, out_hbm,
                    idxa_v, idxr_v, inp_v, am_v, h_v, out_v,
                    sem_i, sem_a, sem_h)

    return _combine


def kernel(f_atoms, f_bonds, a2b, b2a, b2revb, mol_ids, W_i, W_h, W_o, b_o):
    na, fa_dim = f_atoms.shape
    nb = f_bonds.shape[0]
    maxnb = a2b.shape[1]

    a2b_p = jnp.pad(a2b.astype(jnp.int32), ((0, NAP - na), (0, 0)))
    a2b_rs = a2b_p.reshape(NAP * maxnb // GROWS, GROWS)
    aidx = jnp.repeat(jnp.arange(APS, dtype=jnp.int32), maxnb)
    aidx_rs = aidx.reshape(APS * maxnb // GROWS, GROWS)
    nchunks = nb // NW // CB
    b2a_rs = b2a.astype(jnp.int32).reshape(NW, nchunks, CB)
    brev_rs = b2revb.astype(jnp.int32).reshape(NW, nchunks, CB)
    zeros_blk = jnp.zeros((APT, H), jnp.float32)
    ids3 = mol_ids.astype(jnp.int32).reshape(5, 1, na // 5)
    Wo1 = W_o[:fa_dim]
    Wo2 = W_o[fa_dim:]
    bo2 = b_o.reshape(1, H)

    combine = _make_combine(nb)

    inp, msg = _input_matmul(f_bonds, W_i, 1600)

    for _ in range(DEPTH - 1):
        h = _matmul(msg, W_h, 1600)
        ga = _gather_sum(msg, a2b_rs, aidx_rs, zeros_blk)
        am_h = _matmul(ga, W_h, 2048)
        msg = combine(inp, am_h, h, b2a_rs, brev_rs)

    ga = _gather_sum(msg, a2b_rs, aidx_rs, zeros_blk)
    out = _readout(f_atoms, ga[:na], ids3, Wo1, Wo2, bo2, 2000)
    n_mols = 500
    return out[:n_mols]


# speedup vs baseline: 1.2245x; 1.0676x over previous
import functools

import jax
import jax.numpy as jnp
from jax import lax
from jax.experimental import pallas as pl
from jax.experimental.pallas import tpu as pltpu
from jax.experimental.pallas import tpu_sc as plsc

H = 128
DEPTH = 5
NC = 2
NS = 16
NW = NC * NS


def _mm_body(x_ref, w_ref, o_ref):
    o_ref[...] = jnp.dot(x_ref[...], w_ref[...],
                         preferred_element_type=jnp.float32)


def _matmul(x, w, blk):
    m, k = x.shape
    n = w.shape[1]
    return pl.pallas_call(
        _mm_body,
        grid=(m // blk,),
        in_specs=[
            pl.BlockSpec((blk, k), lambda i: (i, 0)),
            pl.BlockSpec((k, n), lambda i: (0, 0)),
        ],
        out_specs=pl.BlockSpec((blk, n), lambda i: (i, 0)),
        out_shape=jax.ShapeDtypeStruct((m, n), jnp.float32),
    )(x, w)


def _mm_relu_body(x_ref, w_ref, inp_ref, msg_ref):
    acc = jnp.dot(x_ref[...], w_ref[...], preferred_element_type=jnp.float32)
    inp_ref[...] = acc
    msg_ref[...] = jnp.maximum(acc, 0.0)


def _input_matmul(f_bonds, W_i, blk):
    m, k = f_bonds.shape
    n = W_i.shape[1]
    shp = jax.ShapeDtypeStruct((m, n), jnp.float32)
    return pl.pallas_call(
        _mm_relu_body,
        grid=(m // blk,),
        in_specs=[
            pl.BlockSpec((blk, k), lambda i: (i, 0)),
            pl.BlockSpec((k, n), lambda i: (0, 0)),
        ],
        out_specs=[
            pl.BlockSpec((blk, n), lambda i: (i, 0)),
            pl.BlockSpec((blk, n), lambda i: (i, 0)),
        ],
        out_shape=[shp, shp],
    )(f_bonds, W_i)


MOLP = 512


def _readout_body(fa_ref, am_ref, ids_ref, wo1_ref, wo2_ref, bo_ref,
                  out_ref, cnt_ref):
    i = pl.program_id(0)
    hidden = jnp.maximum(
        jnp.dot(fa_ref[...], wo1_ref[...], preferred_element_type=jnp.float32)
        + jnp.dot(am_ref[...], wo2_ref[...],
                  preferred_element_type=jnp.float32)
        + bo_ref[...],
        0.0,
    )
    ids = ids_ref[0, 0, :]
    onehot = (lax.broadcasted_iota(jnp.int32, (MOLP, ids.shape[0]), 0)
              == ids[None, :]).astype(jnp.float32)
    part = jnp.dot(onehot, hidden, preferred_element_type=jnp.float32)
    cpart = jnp.sum(onehot, axis=1, keepdims=True)

    @pl.when(i == 0)
    def _():
        out_ref[...] = jnp.zeros_like(out_ref)
        cnt_ref[...] = jnp.zeros_like(cnt_ref)

    out_ref[...] += part
    cnt_ref[...] += jnp.broadcast_to(cpart, cnt_ref.shape)

    @pl.when(i == pl.num_programs(0) - 1)
    def _():
        out_ref[...] = out_ref[...] / jnp.maximum(cnt_ref[...], 1.0)


def _readout(f_atoms, a_msg, ids3, Wo1, Wo2, b_o, blk):
    na, fa = f_atoms.shape
    grid = na // blk
    return pl.pallas_call(
        _readout_body,
        grid=(grid,),
        in_specs=[
            pl.BlockSpec((blk, fa), lambda i: (i, 0)),
            pl.BlockSpec((blk, H), lambda i: (i, 0)),
            pl.BlockSpec((1, 1, blk), lambda i: (i, 0, 0)),
            pl.BlockSpec((fa, H), lambda i: (0, 0)),
            pl.BlockSpec((H, H), lambda i: (0, 0)),
            pl.BlockSpec((1, H), lambda i: (0, 0)),
        ],
        out_specs=pl.BlockSpec((MOLP, H), lambda i: (0, 0)),
        out_shape=jax.ShapeDtypeStruct((MOLP, H), jnp.float32),
        scratch_shapes=[pltpu.VMEM((MOLP, H), jnp.float32)],
    )(f_atoms, a_msg, ids3, Wo1, Wo2, b_o)


_MESH = plsc.VectorSubcoreMesh(core_axis_name="c", subcore_axis_name="s")

NAP = 10240
APS = NAP // NC
APT = APS // NS
GROWS = 128
GCHUNKS = APT * 32 // GROWS


def _sum_chunk(rows, k, acc_v):
    for a in range(4):
        for g in range(H // 16):
            sl = pl.ds(g * 16, 16)
            vals = [rows[a * 32 + r, sl] for r in range(32)]
            while len(vals) > 1:
                vals = [vals[i] + vals[i + 1] for i in range(0, len(vals), 2)]
            acc_v[k * 4 + a, sl] = vals[0]


def _gather_sum_sc(msg_hbm, a2b_hbm, out_hbm,
                   idx_v, rows_a, rows_b, acc_v, sem_a, sem_b):
    c = lax.axis_index("c")
    s = lax.axis_index("s")
    pltpu.sync_copy(
        a2b_hbm.at[pl.ds(c * (APS * 32 // GROWS) + s * GCHUNKS, GCHUNKS)],
        idx_v)
    pltpu.async_copy(msg_hbm.at[idx_v.at[0]], rows_a, sem_a)
    pltpu.async_copy(msg_hbm.at[idx_v.at[1]], rows_b, sem_b)

    def body(j, _):
        k0 = 2 * j
        pltpu.make_async_copy(msg_hbm.at[idx_v.at[k0]], rows_a, sem_a).wait()
        _sum_chunk(rows_a, k0, acc_v)

        @pl.when(k0 + 2 < GCHUNKS)
        def _():
            pltpu.async_copy(msg_hbm.at[idx_v.at[k0 + 2]], rows_a, sem_a)

        pltpu.make_async_copy(msg_hbm.at[idx_v.at[k0 + 1]], rows_b,
                              sem_b).wait()
        _sum_chunk(rows_b, k0 + 1, acc_v)

        @pl.when(k0 + 3 < GCHUNKS)
        def _():
            pltpu.async_copy(msg_hbm.at[idx_v.at[k0 + 3]], rows_b, sem_b)

        return 0

    lax.fori_loop(0, GCHUNKS // 2, body, 0)
    pltpu.sync_copy(acc_v, out_hbm.at[pl.ds(c * APS + s * APT, APT)])


@functools.partial(
    pl.kernel,
    out_type=jax.ShapeDtypeStruct((NAP, H), jnp.float32),
    mesh=_MESH,
    scratch_types=[
        pltpu.VMEM((GCHUNKS, GROWS), jnp.int32),
        pltpu.VMEM((GROWS, H), jnp.float32),
        pltpu.VMEM((GROWS, H), jnp.float32),
        pltpu.VMEM((APT, H), jnp.float32),
        pltpu.SemaphoreType.DMA,
        pltpu.SemaphoreType.DMA,
    ],
)
def _gather_sum(msg_hbm, a2b_hbm, out_hbm,
                idx_v, rows_a, rows_b, acc_v, sem_a, sem_b):
    _gather_sum_sc(msg_hbm, a2b_hbm, out_hbm,
                   idx_v, rows_a, rows_b, acc_v, sem_a, sem_b)


CB = 80


def _combine_sc(nb, inp_hbm, am_hbm, h_hbm, b2a_hbm, brev_hbm, out_hbm,
                idxa_v, idxr_v, inp_v, am_v, h_v, out_v,
                sem_i, sem_a, sem_h):
    c = lax.axis_index("c")
    s = lax.axis_index("s")
    w = s * NC + c
    bpw = nb // NW
    nchunks = bpw // CB
    pltpu.sync_copy(b2a_hbm.at[w], idxa_v)
    pltpu.sync_copy(brev_hbm.at[w], idxr_v)

    def body(j, _):
        base = w * bpw + j * CB
        d_i = pltpu.async_copy(inp_hbm.at[pl.ds(base, CB)], inp_v, sem_i)
        d_a = pltpu.async_copy(am_hbm.at[idxa_v.at[j]], am_v, sem_a)
        d_h = pltpu.async_copy(h_hbm.at[idxr_v.at[j]], h_v, sem_h)
        d_i.wait()
        d_a.wait()
        d_h.wait()

        def row(r, _):
            for g in range(H // 16):
                sl = pl.ds(g * 16, 16)
                out_v[r, sl] = jnp.maximum(
                    inp_v[r, sl] + am_v[r, sl] - h_v[r, sl], 0.0)
            return 0

        lax.fori_loop(0, CB, row, 0)
        pltpu.sync_copy(out_v, out_hbm.at[pl.ds(base, CB)])
        return 0

    lax.fori_loop(0, nchunks, body, 0)


def _make_combine(nb):
    nchunks = nb // NW // CB

    @functools.partial(
        pl.kernel,
        out_type=jax.ShapeDtypeStruct((nb, H), jnp.float32),
        mesh=_MESH,
        scratch_types=[
            pltpu.VMEM((nchunks, CB), jnp.int32),
            pltpu.VMEM((nchunks, CB), jnp.int32),
            pltpu.VMEM((CB, H), jnp.float32),
            pltpu.VMEM((CB, H), jnp.float32),
            pltpu.VMEM((CB, H), jnp.float32),
            pltpu.VMEM((CB, H), jnp.float32),
            pltpu.SemaphoreType.DMA,
            pltpu.SemaphoreType.DMA,
            pltpu.SemaphoreType.DMA,
        ],
    )
    def _combine(inp_hbm, am_hbm, h_hbm, b2a_hbm, brev_hbm, out_hbm,
                 idxa_v, idxr_v, inp_v, am_v, h_v, out_v,
                 sem_i, sem_a, sem_h):
        _combine_sc(nb, inp_hbm, am_hbm, h_hbm, b2a_hbm, brev_hbm, out_hbm,
                    idxa_v, idxr_v, inp_v, am_v, h_v, out_v,
                    sem_i, sem_a, sem_h)

    return _combine


def kernel(f_atoms, f_bonds, a2b, b2a, b2revb, mol_ids, W_i, W_h, W_o, b_o):
    na, fa_dim = f_atoms.shape
    nb = f_bonds.shape[0]
    maxnb = a2b.shape[1]

    a2b_p = jnp.pad(a2b.astype(jnp.int32), ((0, NAP - na), (0, 0)))
    a2b_rs = a2b_p.reshape(NAP * maxnb // GROWS, GROWS)
    nchunks = nb // NW // CB
    b2a_rs = b2a.astype(jnp.int32).reshape(NW, nchunks, CB)
    brev_rs = b2revb.astype(jnp.int32).reshape(NW, nchunks, CB)
    ids3 = mol_ids.astype(jnp.int32).reshape(5, 1, na // 5)
    Wo1 = W_o[:fa_dim]
    Wo2 = W_o[fa_dim:]
    bo2 = b_o.reshape(1, H)

    combine = _make_combine(nb)

    inp, msg = _input_matmul(f_bonds, W_i, 1600)

    for _ in range(DEPTH - 1):
        h = _matmul(msg, W_h, 1600)
        ga = _gather_sum(msg, a2b_rs)
        am_h = _matmul(ga, W_h, 2048)
        msg = combine(inp, am_h, h, b2a_rs, brev_rs)

    ga = _gather_sum(msg, a2b_rs)
    out = _readout(f_atoms, ga[:na], ids3, Wo1, Wo2, bo2, 2000)
    n_mols = 500
    return out[:n_mols]


# speedup vs baseline: 1.2247x; 1.0001x over previous
import functools

import jax
import jax.numpy as jnp
from jax import lax
from jax.experimental import pallas as pl
from jax.experimental.pallas import tpu as pltpu
from jax.experimental.pallas import tpu_sc as plsc

H = 128
DEPTH = 5
NC = 2
NS = 16
NW = NC * NS


def _mm_body(x_ref, w_ref, o_ref):
    o_ref[...] = jnp.dot(x_ref[...], w_ref[...],
                         preferred_element_type=jnp.float32)


def _matmul(x, w, blk):
    m, k = x.shape
    n = w.shape[1]
    return pl.pallas_call(
        _mm_body,
        grid=(m // blk,),
        in_specs=[
            pl.BlockSpec((blk, k), lambda i: (i, 0)),
            pl.BlockSpec((k, n), lambda i: (0, 0)),
        ],
        out_specs=pl.BlockSpec((blk, n), lambda i: (i, 0)),
        out_shape=jax.ShapeDtypeStruct((m, n), jnp.float32),
    )(x, w)


def _mm_relu_body(x_ref, w_ref, inp_ref, msg_ref):
    acc = jnp.dot(x_ref[...], w_ref[...], preferred_element_type=jnp.float32)
    inp_ref[...] = acc
    msg_ref[...] = jnp.maximum(acc, 0.0)


def _input_matmul(f_bonds, W_i, blk):
    m, k = f_bonds.shape
    n = W_i.shape[1]
    shp = jax.ShapeDtypeStruct((m, n), jnp.float32)
    return pl.pallas_call(
        _mm_relu_body,
        grid=(m // blk,),
        in_specs=[
            pl.BlockSpec((blk, k), lambda i: (i, 0)),
            pl.BlockSpec((k, n), lambda i: (0, 0)),
        ],
        out_specs=[
            pl.BlockSpec((blk, n), lambda i: (i, 0)),
            pl.BlockSpec((blk, n), lambda i: (i, 0)),
        ],
        out_shape=[shp, shp],
    )(f_bonds, W_i)


MOLP = 512


def _readout_body(fa_ref, am_ref, ids_ref, wo1_ref, wo2_ref, bo_ref,
                  out_ref, cnt_ref):
    i = pl.program_id(0)
    hidden = jnp.maximum(
        jnp.dot(fa_ref[...], wo1_ref[...], preferred_element_type=jnp.float32)
        + jnp.dot(am_ref[...], wo2_ref[...],
                  preferred_element_type=jnp.float32)
        + bo_ref[...],
        0.0,
    )
    ids = ids_ref[0, 0, :]
    onehot = (lax.broadcasted_iota(jnp.int32, (MOLP, ids.shape[0]), 0)
              == ids[None, :]).astype(jnp.float32)
    part = jnp.dot(onehot, hidden, preferred_element_type=jnp.float32)
    cpart = jnp.sum(onehot, axis=1, keepdims=True)

    @pl.when(i == 0)
    def _():
        out_ref[...] = jnp.zeros_like(out_ref)
        cnt_ref[...] = jnp.zeros_like(cnt_ref)

    out_ref[...] += part
    cnt_ref[...] += jnp.broadcast_to(cpart, cnt_ref.shape)

    @pl.when(i == pl.num_programs(0) - 1)
    def _():
        out_ref[...] = out_ref[...] / jnp.maximum(cnt_ref[...], 1.0)


def _readout(f_atoms, a_msg, ids3, Wo1, Wo2, b_o, blk):
    na, fa = f_atoms.shape
    grid = na // blk
    return pl.pallas_call(
        _readout_body,
        grid=(grid,),
        in_specs=[
            pl.BlockSpec((blk, fa), lambda i: (i, 0)),
            pl.BlockSpec((blk, H), lambda i: (i, 0)),
            pl.BlockSpec((1, 1, blk), lambda i: (i, 0, 0)),
            pl.BlockSpec((fa, H), lambda i: (0, 0)),
            pl.BlockSpec((H, H), lambda i: (0, 0)),
            pl.BlockSpec((1, H), lambda i: (0, 0)),
        ],
        out_specs=pl.BlockSpec((MOLP, H), lambda i: (0, 0)),
        out_shape=jax.ShapeDtypeStruct((MOLP, H), jnp.float32),
        scratch_shapes=[pltpu.VMEM((MOLP, H), jnp.float32)],
    )(f_atoms, a_msg, ids3, Wo1, Wo2, b_o)


_MESH = plsc.VectorSubcoreMesh(core_axis_name="c", subcore_axis_name="s")

NAP = 10240
APS = NAP // NC
APT = APS // NS
GROWS = 128
GCHUNKS = APT * 32 // GROWS


def _sum_chunk(rows, k, acc_v):
    for a in range(4):
        for g in range(H // 16):
            sl = pl.ds(g * 16, 16)
            vals = [rows[a * 32 + r, sl] for r in range(32)]
            while len(vals) > 1:
                vals = [vals[i] + vals[i + 1] for i in range(0, len(vals), 2)]
            acc_v[k * 4 + a, sl] = vals[0]


def _gather_sum_sc(msg_hbm, a2b_hbm, out_hbm,
                   idx_v, rows_a, rows_b, acc_v, sem_a, sem_b):
    c = lax.axis_index("c")
    s = lax.axis_index("s")
    pltpu.sync_copy(
        a2b_hbm.at[pl.ds(c * (APS * 32 // GROWS) + s * GCHUNKS, GCHUNKS)],
        idx_v)
    pltpu.async_copy(msg_hbm.at[idx_v.at[0]], rows_a, sem_a)
    pltpu.async_copy(msg_hbm.at[idx_v.at[1]], rows_b, sem_b)

    def body(j, _):
        k0 = 2 * j
        pltpu.make_async_copy(msg_hbm.at[idx_v.at[k0]], rows_a, sem_a).wait()
        _sum_chunk(rows_a, k0, acc_v)

        @pl.when(k0 + 2 < GCHUNKS)
        def _():
            pltpu.async_copy(msg_hbm.at[idx_v.at[k0 + 2]], rows_a, sem_a)

        pltpu.make_async_copy(msg_hbm.at[idx_v.at[k0 + 1]], rows_b,
                              sem_b).wait()
        _sum_chunk(rows_b, k0 + 1, acc_v)

        @pl.when(k0 + 3 < GCHUNKS)
        def _():
            pltpu.async_copy(msg_hbm.at[idx_v.at[k0 + 3]], rows_b, sem_b)

        return 0

    lax.fori_loop(0, GCHUNKS // 2, body, 0)
    pltpu.sync_copy(acc_v, out_hbm.at[pl.ds(c * APS + s * APT, APT)])


@functools.partial(
    pl.kernel,
    out_type=jax.ShapeDtypeStruct((NAP, H), jnp.float32),
    mesh=_MESH,
    scratch_types=[
        pltpu.VMEM((GCHUNKS, GROWS), jnp.int32),
        pltpu.VMEM((GROWS, H), jnp.float32),
        pltpu.VMEM((GROWS, H), jnp.float32),
        pltpu.VMEM((APT, H), jnp.float32),
        pltpu.SemaphoreType.DMA,
        pltpu.SemaphoreType.DMA,
    ],
)
def _gather_sum(msg_hbm, a2b_hbm, out_hbm,
                idx_v, rows_a, rows_b, acc_v, sem_a, sem_b):
    _gather_sum_sc(msg_hbm, a2b_hbm, out_hbm,
                   idx_v, rows_a, rows_b, acc_v, sem_a, sem_b)


CB = 80


def _combine_sc(nb, inp_hbm, am_hbm, h_hbm, b2a_hbm, brev_hbm, out_hbm,
                idxa_v, idxr_v, inp_v, am_v, h_v, out_v,
                sem_i, sem_a, sem_h):
    c = lax.axis_index("c")
    s = lax.axis_index("s")
    w = s * NC + c
    bpw = nb // NW
    nchunks = bpw // CB
    pltpu.sync_copy(b2a_hbm.at[w], idxa_v)
    pltpu.sync_copy(brev_hbm.at[w], idxr_v)

    def body(j, _):
        base = w * bpw + j * CB
        d_i = pltpu.async_copy(inp_hbm.at[pl.ds(base, CB)], inp_v, sem_i)
        d_a = pltpu.async_copy(am_hbm.at[idxa_v.at[j]], am_v, sem_a)
        d_h = pltpu.async_copy(h_hbm.at[idxr_v.at[j]], h_v, sem_h)
        d_i.wait()
        d_a.wait()
        d_h.wait()

        def row(r, _):
            for g in range(H // 16):
                sl = pl.ds(g * 16, 16)
                out_v[r, sl] = jnp.maximum(
                    inp_v[r, sl] + am_v[r, sl] - h_v[r, sl], 0.0)
            return 0

        lax.fori_loop(0, CB, row, 0)
        pltpu.sync_copy(out_v, out_hbm.at[pl.ds(base, CB)])
        return 0

    lax.fori_loop(0, nchunks, body, 0)


def _make_combine(nb):
    nchunks = nb // NW // CB

    @functools.partial(
        pl.kernel,
        out_type=jax.ShapeDtypeStruct((nb, H), jnp.float32),
        mesh=_MESH,
        scratch_types=[
            pltpu.VMEM((nchunks, CB), jnp.int32),
            pltpu.VMEM((nchunks, CB), jnp.int32),
            pltpu.VMEM((CB, H), jnp.float32),
            pltpu.VMEM((CB, H), jnp.float32),
            pltpu.VMEM((CB, H), jnp.float32),
            pltpu.VMEM((CB, H), jnp.float32),
            pltpu.SemaphoreType.DMA,
            pltpu.SemaphoreType.DMA,
            pltpu.SemaphoreType.DMA,
        ],
    )
    def _combine(inp_hbm, am_hbm, h_hbm, b2a_hbm, brev_hbm, out_hbm,
                 idxa_v, idxr_v, inp_v, am_v, h_v, out_v,
                 sem_i, sem_a, sem_h):
        _combine_sc(nb, inp_hbm, am_hbm, h_hbm, b2a_hbm, brev_hbm, out_hbm,
                    idxa_v, idxr_v, inp_v, am_v, h_v, out_v,
                    sem_i, sem_a, sem_h)

    return _combine


def kernel(f_atoms, f_bonds, a2b, b2a, b2revb, mol_ids, W_i, W_h, W_o, b_o):
    na, fa_dim = f_atoms.shape
    nb = f_bonds.shape[0]
    maxnb = a2b.shape[1]

    a2b_p = jnp.pad(a2b.astype(jnp.int32), ((0, NAP - na), (0, 0)))
    a2b_rs = a2b_p.reshape(NAP * maxnb // GROWS, GROWS)
    nchunks = nb // NW // CB
    b2a_rs = b2a.astype(jnp.int32).reshape(NW, nchunks, CB)
    brev_rs = b2revb.astype(jnp.int32).reshape(NW, nchunks, CB)
    ids3 = mol_ids.astype(jnp.int32).reshape(5, 1, na // 5)
    Wo1 = W_o[:fa_dim]
    Wo2 = W_o[fa_dim:]
    bo2 = b_o.reshape(1, H)

    combine = _make_combine(nb)

    inp, msg = _input_matmul(f_bonds, W_i, 1600)

    for _ in range(DEPTH - 1):
        ga = _gather_sum(msg, a2b_rs)
        h = _matmul(msg, W_h, 1600)
        am_h = _matmul(ga, W_h, 2048)
        msg = combine(inp, am_h, h, b2a_rs, brev_rs)

    ga = _gather_sum(msg, a2b_rs)
    out = _readout(f_atoms, ga[:na], ids3, Wo1, Wo2, bo2, 2000)
    n_mols = 500
    return out[:n_mols]
